# Initial kernel scaffold; baseline (speedup 1.0000x reference)
import functools

import jax
import jax.numpy as jnp
from jax import lax
from jax.experimental import pallas as pl
from jax.experimental.pallas import tpu as pltpu
from jax.experimental.pallas import tpu_sc as plsc

NC = 2
NS = 16
NW = NC * NS
LANES = 16


def _sc_pool_sum(idx2d, table, B, L, E):
    CH_SEGS = 2
    RPC = CH_SEGS * L
    SEGS_PW = B // NW
    NCH = SEGS_PW // CH_SEGS

    mesh = plsc.VectorSubcoreMesh(core_axis_name="c", subcore_axis_name="s")

    @functools.partial(
        pl.kernel,
        out_type=jax.ShapeDtypeStruct((B, E), jnp.float32),
        mesh=mesh,
        scratch_types=[
            pltpu.VMEM((NCH, RPC), jnp.int32),
            pltpu.VMEM((2, RPC, E), jnp.float32),
            pltpu.VMEM((SEGS_PW, E), jnp.float32),
            pltpu.SemaphoreType.DMA,
            pltpu.SemaphoreType.DMA,
        ],
    )
    def sc_kernel(idx_hbm, table_hbm, out_hbm, idx_v, rows_v, pooled_v,
                  sem0, sem1):
        wid = lax.axis_index("s") * NC + lax.axis_index("c")
        base = wid * NCH
        pltpu.sync_copy(idx_hbm.at[pl.ds(base, NCH)], idx_v)

        def start(c, buf, sem):
            pltpu.async_copy(table_hbm.at[idx_v.at[c]], rows_v.at[buf], sem)

        def wait(buf, sem):
            pltpu.make_async_copy(
                table_hbm.at[idx_v.at[0]], rows_v.at[buf], sem).wait()

        zero = jnp.zeros((LANES,), jnp.float32)

        def accum(c, buf):
            for h in range(CH_SEGS):
                rbase = h * L

                def body(r, carry):
                    a0, a1 = carry
                    a0 = a0 + rows_v[buf, rbase + r, pl.ds(0, LANES)]
                    a1 = a1 + rows_v[buf, rbase + r, pl.ds(LANES, LANES)]
                    return a0, a1

                a0, a1 = lax.fori_loop(0, L, body, (zero, zero))
                seg = CH_SEGS * c + h
                pooled_v[seg, pl.ds(0, LANES)] = a0
                pooled_v[seg, pl.ds(LANES, LANES)] = a1

        start(0, 0, sem0)

        @pl.loop(0, NCH, step=2)
        def _(c):
            start(c + 1, 1, sem1)
            wait(0, sem0)
            accum(c, 0)

            @pl.when(c + 2 < NCH)
            def _():
                start(c + 2, 0, sem0)

            wait(1, sem1)
            accum(c + 1, 1)

        pltpu.sync_copy(pooled_v, out_hbm.at[pl.ds(wid * SEGS_PW, SEGS_PW)])

    return sc_kernel(idx2d, table)


def _tc_head(pooled_sum, W, b2d, L, TB=1024):
    B, E = pooled_sum.shape
    C = W.shape[1]
    inv_l = jnp.float32(1.0 / L)

    def body(p_ref, w_ref, b_ref, o_ref):
        pooled = p_ref[...] * inv_l
        logits = jnp.dot(pooled, w_ref[...],
                         preferred_element_type=jnp.float32) + b_ref[...]
        m = jnp.max(logits, axis=1, keepdims=True)
        e = jnp.exp(logits - m)
        o_ref[...] = e / jnp.sum(e, axis=1, keepdims=True)

    return pl.pallas_call(
        body,
        grid=(B // TB,),
        in_specs=[
            pl.BlockSpec((TB, E), lambda i: (i, 0)),
            pl.BlockSpec((E, C), lambda i: (0, 0)),
            pl.BlockSpec((1, C), lambda i: (0, 0)),
        ],
        out_specs=pl.BlockSpec((TB, C), lambda i: (i, 0)),
        out_shape=jax.ShapeDtypeStruct((B, C), jnp.float32),
    )(pooled_sum, W, b2d)


def kernel(x, table, W, b):
    B, L = x.shape
    E = table.shape[1]
    idx2d = x.astype(jnp.int32).reshape(B // 2, 2 * L)
    pooled_sum = _sc_pool_sum(idx2d, table, B, L, E)
    return _tc_head(pooled_sum, W, b.reshape(1, -1), L)


# speedup vs baseline: 2.4666x; 2.4666x over previous
import functools

import jax
import jax.numpy as jnp
from jax import lax
from jax.experimental import pallas as pl
from jax.experimental.pallas import tpu as pltpu
from jax.experimental.pallas import tpu_sc as plsc

NC = 2
NS = 16
NW = NC * NS
LANES = 16


def _sc_pool_sum(idx2d, table, B, L, E):
    CH_SEGS = 2
    RPC = CH_SEGS * L
    SEGS_PW = B // NW
    NCH = SEGS_PW // CH_SEGS

    mesh = plsc.VectorSubcoreMesh(core_axis_name="c", subcore_axis_name="s")

    @functools.partial(
        pl.kernel,
        out_type=jax.ShapeDtypeStruct((B, E), jnp.float32),
        mesh=mesh,
        compiler_params=pltpu.CompilerParams(use_tc_tiling_on_sc=False),
        scratch_types=[
            pltpu.VMEM((NCH, RPC), jnp.int32),
            pltpu.VMEM((2, RPC, E), jnp.float32),
            pltpu.VMEM((SEGS_PW, E), jnp.float32),
            pltpu.SemaphoreType.DMA,
            pltpu.SemaphoreType.DMA,
        ],
    )
    def sc_kernel(idx_hbm, table_hbm, out_hbm, idx_v, rows_v, pooled_v,
                  sem0, sem1):
        wid = lax.axis_index("s") * NC + lax.axis_index("c")
        base = wid * NCH
        pltpu.sync_copy(idx_hbm.at[pl.ds(base, NCH)], idx_v)

        def start(c, buf, sem):
            pltpu.async_copy(table_hbm.at[idx_v.at[c]], rows_v.at[buf], sem)

        def wait(buf, sem):
            pltpu.make_async_copy(
                table_hbm.at[idx_v.at[0]], rows_v.at[buf], sem).wait()

        zero = jnp.zeros((LANES,), jnp.float32)

        def accum(c, buf):
            for h in range(CH_SEGS):
                rbase = h * L

                def body(r, carry):
                    a0, a1 = carry
                    a0 = a0 + rows_v[buf, rbase + r, pl.ds(0, LANES)]
                    a1 = a1 + rows_v[buf, rbase + r, pl.ds(LANES, LANES)]
                    return a0, a1

                a0, a1 = lax.fori_loop(0, L, body, (zero, zero))
                seg = CH_SEGS * c + h
                pooled_v[seg, pl.ds(0, LANES)] = a0
                pooled_v[seg, pl.ds(LANES, LANES)] = a1

        start(0, 0, sem0)

        @pl.loop(0, NCH, step=2)
        def _(c):
            start(c + 1, 1, sem1)
            wait(0, sem0)
            accum(c, 0)

            @pl.when(c + 2 < NCH)
            def _():
                start(c + 2, 0, sem0)

            wait(1, sem1)
            accum(c + 1, 1)

        pltpu.sync_copy(pooled_v, out_hbm.at[pl.ds(wid * SEGS_PW, SEGS_PW)])

    return sc_kernel(idx2d, table)


def _tc_head(pooled_sum, W, b2d, L, TB=1024):
    B, E = pooled_sum.shape
    C = W.shape[1]
    inv_l = 1.0 / L

    def body(p_ref, w_ref, b_ref, o_ref):
        pooled = p_ref[...] * inv_l
        logits = jnp.dot(pooled, w_ref[...],
                         preferred_element_type=jnp.float32) + b_ref[...]
        m = jnp.max(logits, axis=1, keepdims=True)
        e = jnp.exp(logits - m)
        o_ref[...] = e / jnp.sum(e, axis=1, keepdims=True)

    return pl.pallas_call(
        body,
        grid=(B // TB,),
        in_specs=[
            pl.BlockSpec((TB, E), lambda i: (i, 0)),
            pl.BlockSpec((E, C), lambda i: (0, 0)),
            pl.BlockSpec((1, C), lambda i: (0, 0)),
        ],
        out_specs=pl.BlockSpec((TB, C), lambda i: (i, 0)),
        out_shape=jax.ShapeDtypeStruct((B, C), jnp.float32),
    )(pooled_sum, W, b2d)


def kernel(x, table, W, b):
    B, L = x.shape
    E = table.shape[1]
    idx2d = x.astype(jnp.int32).reshape(B // 2, 2 * L)
    pooled_sum = _sc_pool_sum(idx2d, table, B, L, E)
    return _tc_head(pooled_sum, W, b.reshape(1, -1), L)
